# baseline (speedup 1.0000x reference)
import functools

import jax
import jax.numpy as jnp
from jax import lax
from jax.experimental import pallas as pl
from jax.experimental.pallas import tpu as pltpu
from jax.experimental.pallas import tpu_sc as plsc

_V = 1000
_B = 16
_T = 2048
_N = _B * _T

_NC = 2
_NS = 16
_NW = _NC * _NS
_L = 16
_PER_W = _N // _NW
_C = 64
_NCHUNK = _PER_W // _C


def _lse_tc_kernel(table_ref, lse_ref):
    t = table_ref[...]
    m = jnp.max(t, axis=1, keepdims=True)
    s = jnp.sum(jnp.exp(t - m), axis=1, keepdims=True)
    lse_ref[...] = (m + jnp.log(s)).reshape(_V)


@jax.jit
def _lse_rows(table):
    return pl.pallas_call(
        _lse_tc_kernel,
        out_shape=jax.ShapeDtypeStruct((_V,), jnp.float32),
    )(table)


def _sc_body(ctx_hbm, tgt_hbm, table_hbm, lse_hbm,
             out_hbm, part_hbm,
             ctx_v, tgt_v, lse_v, buf0, buf1, acc_v,
             sg0, sg1, sw0, sw1):
    wid = lax.axis_index("s") * _NC + lax.axis_index("c")
    base = wid * _PER_W
    bi = base // _T
    r0 = base % _T

    pltpu.sync_copy(ctx_hbm.at[pl.ds(base, _PER_W)], ctx_v)
    pltpu.sync_copy(tgt_hbm.at[pl.ds(base, _PER_W)], tgt_v)
    pltpu.sync_copy(lse_hbm, lse_v)

    bufs, sgs, sws = (buf0, buf1), (sg0, sg1), (sw0, sw1)

    def start_gather(g, b):
        pltpu.async_copy(table_hbm.at[ctx_v.at[pl.ds(g * _C, _C)]],
                         bufs[b], sgs[b])

    def wait_gather(b):
        pltpu.make_async_copy(table_hbm.at[ctx_v.at[pl.ds(0, _C)]],
                              bufs[b], sgs[b]).wait()

    def wait_write(b):
        pltpu.make_async_copy(bufs[b], out_hbm.at[0, pl.ds(0, _C)],
                              sws[b]).wait()

    def loss_chunk(buf, g, acc):
        for j in range(_C // _L):
            off = g * _C + j * _L
            c16 = ctx_v[pl.ds(off, _L)]
            t16 = tgt_v[pl.ds(off, _L)]
            r16 = lax.iota(jnp.int32, _L) + j * _L
            picked = plsc.load_gather(buf, [r16, t16])
            l16 = plsc.load_gather(lse_v, [c16])
            acc = acc + (l16 - picked)
        return acc

    start_gather(0, 0)
    start_gather(1, 1)

    def body(i, acc):
        g2 = 2 * i
        for b in range(2):
            g = g2 + b
            wait_gather(b)
            pltpu.async_copy(bufs[b],
                             out_hbm.at[bi, pl.ds(r0 + g * _C, _C)],
                             sws[b])
            acc = loss_chunk(bufs[b], g, acc)

            @pl.when(g + 2 < _NCHUNK)
            def _():
                wait_write(b)
                start_gather(g + 2, b)
        return acc

    acc = lax.fori_loop(0, _NCHUNK // 2, body,
                        jnp.zeros((_L,), jnp.float32))
    wait_write(0)
    wait_write(1)
    acc_v[...] = acc
    pltpu.sync_copy(acc_v, part_hbm.at[wid])


@jax.jit
def _sc_call(ctx_flat, tgt_flat, table, lse):
    fn = pl.kernel(
        _sc_body,
        out_type=(
            jax.ShapeDtypeStruct((_B, _T, _V), jnp.float32),
            jax.ShapeDtypeStruct((_NW, _L), jnp.float32),
        ),
        mesh=plsc.VectorSubcoreMesh(core_axis_name="c", subcore_axis_name="s"),
        compiler_params=pltpu.CompilerParams(
            use_tc_tiling_on_sc=False, needs_layout_passes=False),
        scratch_types=(
            pltpu.VMEM((_PER_W,), jnp.int32),
            pltpu.VMEM((_PER_W,), jnp.int32),
            pltpu.VMEM((_V,), jnp.float32),
            pltpu.VMEM((_C, _V), jnp.float32),
            pltpu.VMEM((_C, _V), jnp.float32),
            pltpu.VMEM((_L,), jnp.float32),
            pltpu.SemaphoreType.DMA,
            pltpu.SemaphoreType.DMA,
            pltpu.SemaphoreType.DMA,
            pltpu.SemaphoreType.DMA,
        ),
    )
    return fn(ctx_flat, tgt_flat, table, lse)


def kernel(contexts, targets, table):
    ctx_flat = contexts.reshape(_N)
    tgt_flat = targets.reshape(_N)
    lse = _lse_rows(table)
    logits, partials = _sc_call(ctx_flat, tgt_flat, table, lse)
    loss = jnp.sum(partials) / jnp.float32(_N)
    return (logits, loss)


# speedup vs baseline: 2.4640x; 2.4640x over previous
import functools

import jax
import jax.numpy as jnp
from jax import lax
from jax.experimental import pallas as pl
from jax.experimental.pallas import tpu as pltpu
from jax.experimental.pallas import tpu_sc as plsc

_V = 1000
_B = 16
_T = 2048
_N = _B * _T

_NC = 2
_NS = 16
_NW = _NC * _NS
_L = 16
_PER_W = _N // _NW
_C = 32
_NCHUNK = _PER_W // _C
_VT = _V // 8


def _lse_tc_kernel(table_ref, lse_ref):
    t = table_ref[...]
    m = jnp.max(t, axis=1, keepdims=True)
    s = jnp.sum(jnp.exp(t - m), axis=1, keepdims=True)
    lse_ref[...] = (m + jnp.log(s)).reshape(_V)


@jax.jit
def _lse_rows(table):
    return pl.pallas_call(
        _lse_tc_kernel,
        out_shape=jax.ShapeDtypeStruct((_V,), jnp.float32),
    )(table)


def _sc_body(ctx_hbm, tgt_hbm, table_hbm, lse_hbm,
             out_hbm, part_hbm,
             ctx_v, tgt_v, lse_v, buf0, buf1, bufT0, bufT1,
             sg0, sg1, sw0, sw1):
    wid = lax.axis_index("s") * _NC + lax.axis_index("c")
    base = wid * _PER_W
    bi = base // _T
    tt0 = (base % _T) // 128

    pltpu.sync_copy(ctx_hbm.at[pl.ds(base, _PER_W)], ctx_v)
    pltpu.sync_copy(tgt_hbm.at[pl.ds(base, _PER_W)], tgt_v)
    pltpu.sync_copy(lse_hbm, lse_v)

    bufs, bufTs, sgs, sws = (buf0, buf1), (bufT0, bufT1), (sg0, sg1), (sw0, sw1)

    def start_gather(g, b):
        pltpu.async_copy(table_hbm.at[ctx_v.at[pl.ds(g * _C, _C)]],
                         bufs[b], sgs[b])

    def wait_gather(b):
        pltpu.make_async_copy(table_hbm.at[ctx_v.at[pl.ds(0, _C)]],
                              bufs[b], sgs[b]).wait()

    def out_slab(g):
        tt = tt0 + g // 4
        l0 = (g % 4) * _C
        return out_hbm.at[bi, pl.ds(0, _VT), tt, pl.ds(0, 8), pl.ds(l0, _C)]

    def wait_write(b):
        pltpu.make_async_copy(bufTs[b], out_slab(0), sws[b]).wait()

    def transpose_chunk(buf, bufT):
        @plsc.parallel_loop(0, _VT, unroll=6)
        def vt_body(vt):
            for s in range(8):
                v16 = jnp.full((_L,), vt * 8 + s, jnp.int32)
                for q in range(_C // _L):
                    l16 = lax.iota(jnp.int32, _L) + q * _L
                    vals = plsc.load_gather(buf, [l16, v16])
                    bufT[vt, s, pl.ds(q * _L, _L)] = vals

    def loss_chunk(buf, g, acc):
        for j in range(_C // _L):
            off = g * _C + j * _L
            c16 = ctx_v[pl.ds(off, _L)]
            t16 = tgt_v[pl.ds(off, _L)]
            r16 = lax.iota(jnp.int32, _L) + j * _L
            picked = plsc.load_gather(buf, [r16, t16])
            l16 = plsc.load_gather(lse_v, [c16])
            acc = acc + (l16 - picked)
        return acc

    start_gather(0, 0)
    start_gather(1, 1)

    def body(i, acc):
        g2 = 2 * i
        for b in range(2):
            g = g2 + b
            wait_gather(b)

            acc = loss_chunk(bufs[b], g, acc)

            @pl.when(g >= 2)
            def _():
                wait_write(b)
            transpose_chunk(bufs[b], bufTs[b])

            @pl.when(g + 2 < _NCHUNK)
            def _():
                start_gather(g + 2, b)
            pltpu.async_copy(bufTs[b], out_slab(g), sws[b])
        return acc

    acc = lax.fori_loop(0, _NCHUNK // 2, body,
                        jnp.zeros((_L,), jnp.float32))
    wait_write(0)
    wait_write(1)
    bufT0[0, 0, pl.ds(0, _L)] = acc
    pltpu.sync_copy(bufT0.at[0, 0, pl.ds(0, _L)], part_hbm.at[wid])


@jax.jit
def _sc_call(ctx_flat, tgt_flat, table, lse):
    fn = pl.kernel(
        _sc_body,
        out_type=(
            jax.ShapeDtypeStruct((_B, _VT, _T // 128, 8, 128), jnp.float32),
            jax.ShapeDtypeStruct((_NW, _L), jnp.float32),
        ),
        mesh=plsc.VectorSubcoreMesh(core_axis_name="c", subcore_axis_name="s"),
        compiler_params=pltpu.CompilerParams(
            use_tc_tiling_on_sc=False, needs_layout_passes=False),
        scratch_types=(
            pltpu.VMEM((_PER_W,), jnp.int32),
            pltpu.VMEM((_PER_W,), jnp.int32),
            pltpu.VMEM((_V,), jnp.float32),
            pltpu.VMEM((_C, _V), jnp.float32),
            pltpu.VMEM((_C, _V), jnp.float32),
            pltpu.VMEM((_VT, 8, _C), jnp.float32),
            pltpu.VMEM((_VT, 8, _C), jnp.float32),
            pltpu.SemaphoreType.DMA,
            pltpu.SemaphoreType.DMA,
            pltpu.SemaphoreType.DMA,
            pltpu.SemaphoreType.DMA,
        ),
    )
    return fn(ctx_flat, tgt_flat, table, lse)


def kernel(contexts, targets, table):
    ctx_flat = contexts.reshape(_N)
    tgt_flat = targets.reshape(_N)
    lse = _lse_rows(table)
    out5, partials = _sc_call(ctx_flat, tgt_flat, table, lse)
    logits = out5.transpose(0, 2, 4, 1, 3).reshape(_B, _T, _V)
    loss = jnp.sum(partials) / jnp.float32(_N)
    return (logits, loss)
